# baseline (speedup 1.0000x reference)
import jax
import jax.numpy as jnp
from jax import lax
from jax.experimental import pallas as pl
from jax.experimental.pallas import tpu as pltpu
from jax.experimental.pallas import tpu_sc as plsc

HOTN = 32768
HASH_SIZE = 500000
EMB_DIM = 32
BATCH = 16384

_NC = 2
_NS = 16
_NW = _NC * _NS
_BPW = BATCH // _NW
_NVEC = _BPW // 16
_CH = 128
_NCH = _BPW // _CH


def _sc_body(inp_hbm, wh_hbm, whash_hbm, out_hbm,
             raw_v, gh_v, gc_v, offh_v, offc_v, maskf_v,
             hot_b, cold_b, out_b, sem):
    wid = lax.axis_index("s") * _NC + lax.axis_index("c")
    base = wid * _BPW

    pltpu.sync_copy(inp_hbm.at[pl.ds(base, _BPW)], raw_v)

    for i in range(_NVEC):
        v = raw_v[pl.ds(i * 16, 16)]
        s = (v & 31) + ((v >> 5) & 31) + ((v >> 10) & 31) + ((v >> 15) & 31)
        s = (s & 31) + (s >> 5)
        hot = jnp.logical_or(s == 0, s == 31)
        maskf_v[pl.ds(i * 16, 16)] = jnp.where(hot, 1.0, 0.0).astype(jnp.float32)
        rh = v & (HOTN - 1)
        rc = jnp.where(v >= HASH_SIZE, v - HASH_SIZE, v)
        gh_v[i // 8, pl.ds((i % 8) * 16, 16)] = rh >> 2
        gc_v[i // 8, pl.ds((i % 8) * 16, 16)] = rc >> 2
        offh_v[pl.ds(i * 16, 16)] = (rh & 3) << 5
        offc_v[pl.ds(i * 16, 16)] = (rc & 3) << 5

    for ch in range(_NCH):
        c1 = pltpu.async_copy(wh_hbm.at[gh_v.at[ch]], hot_b, sem)
        c2 = pltpu.async_copy(whash_hbm.at[gc_v.at[ch]], cold_b, sem)
        c1.wait()
        c2.wait()

        def blend(blk, _):
            b16 = ch * _CH + blk * 16
            offh16 = offh_v[pl.ds(b16, 16)]
            offc16 = offc_v[pl.ds(b16, 16)]
            m16 = maskf_v[pl.ds(b16, 16)]
            for jj in range(16):
                oh = offh16[jj]
                oc = offc16[jj]
                m = m16[jj]
                il = blk * 16 + jj
                orow = blk * 4 + (jj >> 2)
                ocol = (jj & 3) * 32
                for c0 in (0, 16):
                    h = hot_b[il, pl.ds(oh + c0, 16)]
                    g = cold_b[il, pl.ds(oc + c0, 16)]
                    out_b[orow, pl.ds(ocol + c0, 16)] = g + m * (h - g)
            return 0

        lax.fori_loop(0, _CH // 16, blend, 0)
        pltpu.sync_copy(out_b, out_hbm.at[pl.ds(wid * 128 + ch * 32, 32)])


@jax.jit
def _run(inp, wh, whash):
    mesh = plsc.VectorSubcoreMesh(core_axis_name="c", subcore_axis_name="s")
    f = pl.kernel(
        _sc_body,
        out_type=jax.ShapeDtypeStruct((BATCH // 4, 128), jnp.float32),
        mesh=mesh,
        compiler_params=pltpu.CompilerParams(use_tc_tiling_on_sc=True),
        scratch_types=[
            pltpu.VMEM((_BPW,), jnp.int32),
            pltpu.VMEM((_NCH, _CH), jnp.int32),
            pltpu.VMEM((_NCH, _CH), jnp.int32),
            pltpu.VMEM((_BPW,), jnp.int32),
            pltpu.VMEM((_BPW,), jnp.int32),
            pltpu.VMEM((_BPW,), jnp.float32),
            pltpu.VMEM((_CH, 128), jnp.float32),
            pltpu.VMEM((_CH, 128), jnp.float32),
            pltpu.VMEM((32, 128), jnp.float32),
            pltpu.SemaphoreType.DMA,
        ],
    )
    return f(inp, wh, whash)


def kernel(input, offsets, weight_h, weight_hash):
    del offsets
    wh = weight_h.reshape(HOTN // 4, 128)
    whash = weight_hash.reshape(HASH_SIZE // 4, 128)
    out = _run(input.astype(jnp.int32), wh, whash)
    return out.reshape(BATCH, EMB_DIM)
